# baseline (speedup 1.0000x reference)
import functools

import jax
import jax.numpy as jnp
from jax import lax
from jax.experimental import pallas as pl
from jax.experimental.pallas import tpu as pltpu
from jax.experimental.pallas import tpu_sc as plsc

_BATCH, _SEQ = 4096, 200
_HALF = _BATCH // 4
_VOCAB, _ADD_VOCAB, _DIM = 100000, 1024, 64
_NC, _NS, _L = 2, 16, 16
_NW = _NC * _NS
_RPW = _HALF // _NW
_RPC = 4
_CHUNK = _RPC * _SEQ
_NCHUNK = _RPW // _RPC
_NGROUP = _CHUNK // _L
_IDXW = 80
_NGATH = _CHUNK // _IDXW
_NBUF = 2
_PEEL = 2
_NSTEADY = _NCHUNK - _PEEL

_mesh = plsc.VectorSubcoreMesh(
    core_axis_name="c", subcore_axis_name="s", num_cores=_NC, num_subcores=_NS
)


def _make_sc(half):
  @functools.partial(
    pl.kernel,
    out_type=jax.ShapeDtypeStruct((_HALF, _SEQ, _DIM), jnp.float32),
    mesh=_mesh,
    compiler_params=pltpu.CompilerParams(
        use_tc_tiling_on_sc=False, needs_layout_passes=False
    ),
    scratch_types=(
        [pltpu.VMEM((_CHUNK,), jnp.int32) for _ in range(_NBUF)]
        + [pltpu.VMEM((_CHUNK,), jnp.int32) for _ in range(_NBUF)]
        + [pltpu.VMEM((_CHUNK,), jnp.int32) for _ in range(_NBUF)]
        + [pltpu.VMEM((_CHUNK, _DIM), jnp.float32) for _ in range(_NBUF)]
        + [pltpu.SMEM((_NBUF,), jnp.int32)]
        + [pltpu.VMEM((_L,), jnp.int32)]
        + [pltpu.VMEM((_L, _DIM), jnp.float32)]
        + [pltpu.SemaphoreType.DMA for _ in range(4 * _NBUF + 1)]
    ),
  )
  def _sc_lookup(iid_hbm, aid_hbm, base_hbm, add_hbm, out_hbm, *scratch):
    iid_v = scratch[0:_NBUF]
    aid_v = scratch[_NBUF:2 * _NBUF]
    idx_v = scratch[2 * _NBUF:3 * _NBUF]
    rows_v = scratch[3 * _NBUF:4 * _NBUF]
    nneg_s = scratch[4 * _NBUF]
    pidx_v = scratch[4 * _NBUF + 1]
    prow_v = scratch[4 * _NBUF + 2]
    sems = scratch[4 * _NBUF + 3:]
    sem_i = sems[0:_NBUF]
    sem_a = sems[_NBUF:2 * _NBUF]
    sem_g = sems[2 * _NBUF:3 * _NBUF]
    sem_w = sems[3 * _NBUF:4 * _NBUF]
    sem_p = sems[4 * _NBUF]

    wid = lax.axis_index("s") * _NC + lax.axis_index("c")
    wrow = wid * _RPW
    growbase = half * _HALF + wrow

    def start_ids(c, b):
        row0 = jnp.minimum(growbase + c * _RPC, _BATCH - _RPC)
        for r in range(_RPC):
            pltpu.async_copy(iid_hbm.at[row0 + r],
                             iid_v[b].at[pl.ds(r * _SEQ, _SEQ)], sem_i[b])
            pltpu.async_copy(aid_hbm.at[row0 + r],
                             aid_v[b].at[pl.ds(r * _SEQ, _SEQ)], sem_a[b])

    def wait_ids(b):
        for r in range(_RPC):
            pltpu.make_async_copy(iid_hbm.at[0],
                                  iid_v[b].at[pl.ds(r * _SEQ, _SEQ)],
                                  sem_i[b]).wait()
            pltpu.make_async_copy(aid_hbm.at[0],
                                  aid_v[b].at[pl.ds(r * _SEQ, _SEQ)],
                                  sem_a[b]).wait()

    def compute_idx(b):
        acc = jnp.zeros((_L,), jnp.int32)
        one = jnp.ones((_L,), jnp.int32)
        zero = jnp.zeros((_L,), jnp.int32)
        for g in range(_NGROUP):
            sl = pl.ds(g * _L, _L)
            a = aid_v[b][sl]
            idx_v[b][sl] = jnp.maximum(a, 0)
            acc = acc + jnp.where(a == -1, one, zero)
        nneg_s[b] = jnp.sum(acc)

    def fire_gathers(b):
        for j in range(_NGATH):
            pltpu.async_copy(
                add_hbm.at[idx_v[b].at[pl.ds(j * _IDXW, _IDXW)]],
                rows_v[b].at[pl.ds(j * _IDXW, _IDXW), :],
                sem_g[b],
            )

    def wait_gathers(b):
        for j in range(_NGATH):
            pltpu.make_async_copy(
                add_hbm.at[idx_v[b].at[pl.ds(j * _IDXW, _IDXW)]],
                rows_v[b].at[pl.ds(j * _IDXW, _IDXW), :],
                sem_g[b],
            ).wait()

    def patch(b):
        @pl.when(nneg_s[b] != 0)
        def _slow():
            lane = lax.iota(jnp.int32, _L)

            def grp(g, carry):
                sl = pl.ds(g * _L, _L)
                a = aid_v[b][sl]
                i = iid_v[b][sl]
                mask = a == -1
                pidx_v[...] = jnp.where(mask, i, 0)
                pltpu.async_copy(base_hbm.at[pidx_v], prow_v, sem_p).wait()
                tok = g * _L + lane
                for p in range(_DIM):
                    col = jnp.full((_L,), p, jnp.int32)
                    vals = plsc.load_gather(prow_v, [lane, col])
                    plsc.store_scatter(rows_v[b], [tok, col], vals, mask=mask)
                return carry

            lax.fori_loop(0, _NGROUP, grp, 0)

    def start_wb(c, b):
        row0 = wrow + c * _RPC
        for r in range(_RPC):
            pltpu.async_copy(rows_v[b].at[pl.ds(r * _SEQ, _SEQ), :],
                             out_hbm.at[row0 + r], sem_w[b])

    def wait_wb(b):
        for r in range(_RPC):
            pltpu.make_async_copy(rows_v[b].at[pl.ds(r * _SEQ, _SEQ), :],
                                  out_hbm.at[0], sem_w[b]).wait()

    start_ids(0, 0)
    start_ids(1, 1)
    wait_ids(0); compute_idx(0); fire_gathers(0); start_ids(2, 0)
    wait_ids(1); compute_idx(1); fire_gathers(1)
    wait_gathers(0); patch(0); start_wb(0, 0); start_ids(3, 1)

    def steady(i, carry):
        for k in range(_NBUF):
            cc = _PEEL + i * _NBUF + k
            b = k
            bp = 1 - k
            wait_ids(b)
            compute_idx(b)
            wait_wb(b)
            fire_gathers(b)
            wait_gathers(bp)
            patch(bp)
            start_wb(cc - 1, bp)
            start_ids(cc + 2, b)
        return carry

    lax.fori_loop(0, _NSTEADY // _NBUF, steady, 0)

    last = _NCHUNK - 1
    bl = last % _NBUF
    wait_gathers(bl)
    patch(bl)
    start_wb(last, bl)
    wait_ids(_NCHUNK % _NBUF)
    wait_ids((_NCHUNK + 1) % _NBUF)
    for b in range(_NBUF):
        wait_wb(b)

  return _sc_lookup


_sc_stage = [_make_sc(h) for h in range(_BATCH // _HALF)]

_BB = 128
_HR = _SEQ * _DIM // 128
_NBLK = _HALF // _BB


def _retile_body(flat_ref, out_ref):
    x = flat_ref[...].reshape(_BB, _HR, 128)
    a = x[:, :, :_DIM]
    b = x[:, :, _DIM:]
    out_ref[pl.Slice(0, _HR, 2), :, :] = jnp.transpose(a, (1, 2, 0))
    out_ref[pl.Slice(1, _HR, 2), :, :] = jnp.transpose(b, (1, 2, 0))


_retile_first = functools.partial(
    pl.pallas_call,
    grid=(_NBLK,),
    in_specs=[pl.BlockSpec((_BB * _HR, 128), lambda i: (i, 0))],
    out_specs=pl.BlockSpec((_SEQ, _DIM, _BB), lambda i: (0, 0, i)),
    out_shape=jax.ShapeDtypeStruct((_SEQ, _DIM, _BATCH), jnp.float32),
)(_retile_body)


def _retile_next_body(flat_ref, prev_ref, out_ref):
    del prev_ref
    _retile_body(flat_ref, out_ref)


def _make_retile_next(stage):
    off = stage * _NBLK
    return functools.partial(
        pl.pallas_call,
        grid=(_NBLK,),
        in_specs=[
            pl.BlockSpec((_BB * _HR, 128), lambda i: (i, 0)),
            pl.BlockSpec(memory_space=pl.ANY),
        ],
        out_specs=pl.BlockSpec(
            (_SEQ, _DIM, _BB), lambda i, _o=off: (0, 0, i + _o)
        ),
        out_shape=jax.ShapeDtypeStruct((_SEQ, _DIM, _BATCH), jnp.float32),
        input_output_aliases={1: 0},
    )(_retile_next_body)


_retile_next = [_make_retile_next(s) for s in range(1, _BATCH // _HALF)]


def kernel(input_ids, additional_token_ids, base_table, add_table):
    iid = input_ids.astype(jnp.int32)
    aid = additional_token_ids.astype(jnp.int32)
    lins = [sc(iid, aid, base_table, add_table) for sc in _sc_stage]
    flat = [l.reshape(_HALF * _SEQ * _DIM // 128, 128) for l in lins]
    out = _retile_first(flat[0])
    for s in range(1, _BATCH // _HALF):
        out = _retile_next[s - 1](flat[s], out)
    return out.transpose(2, 0, 1)


# speedup vs baseline: 1.0462x; 1.0462x over previous
import functools

import jax
import jax.numpy as jnp
from jax import lax
from jax.experimental import pallas as pl
from jax.experimental.pallas import tpu as pltpu
from jax.experimental.pallas import tpu_sc as plsc

_BATCH, _SEQ = 4096, 200
_HALF = _BATCH // 2
_VOCAB, _ADD_VOCAB, _DIM = 100000, 1024, 64
_NC, _NS, _L = 2, 16, 16
_NW = _NC * _NS
_RPW = _HALF // _NW
_RPC = 4
_CHUNK = _RPC * _SEQ
_NCHUNK = _RPW // _RPC
_NGROUP = _CHUNK // _L
_IDXW = 80
_NGATH = _CHUNK // _IDXW
_NBUF = 2
_PEEL = 2
_NSTEADY = _NCHUNK - _PEEL

_mesh = plsc.VectorSubcoreMesh(
    core_axis_name="c", subcore_axis_name="s", num_cores=_NC, num_subcores=_NS
)


def _make_sc(half):
  @functools.partial(
    pl.kernel,
    out_type=jax.ShapeDtypeStruct((_HALF, _SEQ, _DIM), jnp.float32),
    mesh=_mesh,
    compiler_params=pltpu.CompilerParams(
        use_tc_tiling_on_sc=False, needs_layout_passes=False
    ),
    scratch_types=(
        [pltpu.VMEM((_CHUNK,), jnp.int32) for _ in range(_NBUF)]
        + [pltpu.VMEM((_CHUNK,), jnp.int32) for _ in range(_NBUF)]
        + [pltpu.VMEM((_CHUNK,), jnp.int32) for _ in range(_NBUF)]
        + [pltpu.VMEM((_CHUNK, _DIM), jnp.float32) for _ in range(_NBUF)]
        + [pltpu.SMEM((_NBUF,), jnp.int32)]
        + [pltpu.VMEM((_L,), jnp.int32)]
        + [pltpu.VMEM((_L, _DIM), jnp.float32)]
        + [pltpu.SemaphoreType.DMA for _ in range(4 * _NBUF + 1)]
    ),
  )
  def _sc_lookup(iid_hbm, aid_hbm, base_hbm, add_hbm, out_hbm, *scratch):
    iid_v = scratch[0:_NBUF]
    aid_v = scratch[_NBUF:2 * _NBUF]
    idx_v = scratch[2 * _NBUF:3 * _NBUF]
    rows_v = scratch[3 * _NBUF:4 * _NBUF]
    nneg_s = scratch[4 * _NBUF]
    pidx_v = scratch[4 * _NBUF + 1]
    prow_v = scratch[4 * _NBUF + 2]
    sems = scratch[4 * _NBUF + 3:]
    sem_i = sems[0:_NBUF]
    sem_a = sems[_NBUF:2 * _NBUF]
    sem_g = sems[2 * _NBUF:3 * _NBUF]
    sem_w = sems[3 * _NBUF:4 * _NBUF]
    sem_p = sems[4 * _NBUF]

    wid = lax.axis_index("s") * _NC + lax.axis_index("c")
    wrow = wid * _RPW
    growbase = half * _HALF + wrow

    def start_ids(c, b):
        row0 = jnp.minimum(growbase + c * _RPC, _BATCH - _RPC)
        for r in range(_RPC):
            pltpu.async_copy(iid_hbm.at[row0 + r],
                             iid_v[b].at[pl.ds(r * _SEQ, _SEQ)], sem_i[b])
            pltpu.async_copy(aid_hbm.at[row0 + r],
                             aid_v[b].at[pl.ds(r * _SEQ, _SEQ)], sem_a[b])

    def wait_ids(b):
        for r in range(_RPC):
            pltpu.make_async_copy(iid_hbm.at[0],
                                  iid_v[b].at[pl.ds(r * _SEQ, _SEQ)],
                                  sem_i[b]).wait()
            pltpu.make_async_copy(aid_hbm.at[0],
                                  aid_v[b].at[pl.ds(r * _SEQ, _SEQ)],
                                  sem_a[b]).wait()

    def compute_idx(b):
        acc = jnp.zeros((_L,), jnp.int32)
        one = jnp.ones((_L,), jnp.int32)
        zero = jnp.zeros((_L,), jnp.int32)
        for g in range(_NGROUP):
            sl = pl.ds(g * _L, _L)
            a = aid_v[b][sl]
            idx_v[b][sl] = jnp.maximum(a, 0)
            acc = acc + jnp.where(a == -1, one, zero)
        nneg_s[b] = jnp.sum(acc)

    def fire_gathers(b):
        for j in range(_NGATH):
            pltpu.async_copy(
                add_hbm.at[idx_v[b].at[pl.ds(j * _IDXW, _IDXW)]],
                rows_v[b].at[pl.ds(j * _IDXW, _IDXW), :],
                sem_g[b],
            )

    def wait_gathers(b):
        for j in range(_NGATH):
            pltpu.make_async_copy(
                add_hbm.at[idx_v[b].at[pl.ds(j * _IDXW, _IDXW)]],
                rows_v[b].at[pl.ds(j * _IDXW, _IDXW), :],
                sem_g[b],
            ).wait()

    def patch(b):
        @pl.when(nneg_s[b] != 0)
        def _slow():
            lane = lax.iota(jnp.int32, _L)

            def grp(g, carry):
                sl = pl.ds(g * _L, _L)
                a = aid_v[b][sl]
                i = iid_v[b][sl]
                mask = a == -1
                pidx_v[...] = jnp.where(mask, i, 0)
                pltpu.async_copy(base_hbm.at[pidx_v], prow_v, sem_p).wait()
                tok = g * _L + lane
                for p in range(_DIM):
                    col = jnp.full((_L,), p, jnp.int32)
                    vals = plsc.load_gather(prow_v, [lane, col])
                    plsc.store_scatter(rows_v[b], [tok, col], vals, mask=mask)
                return carry

            lax.fori_loop(0, _NGROUP, grp, 0)

    def start_wb(c, b):
        row0 = wrow + c * _RPC
        for r in range(_RPC):
            pltpu.async_copy(rows_v[b].at[pl.ds(r * _SEQ, _SEQ), :],
                             out_hbm.at[row0 + r], sem_w[b])

    def wait_wb(b):
        for r in range(_RPC):
            pltpu.make_async_copy(rows_v[b].at[pl.ds(r * _SEQ, _SEQ), :],
                                  out_hbm.at[0], sem_w[b]).wait()

    start_ids(0, 0)
    start_ids(1, 1)
    wait_ids(0); compute_idx(0); fire_gathers(0); start_ids(2, 0)
    wait_ids(1); compute_idx(1); fire_gathers(1)
    wait_gathers(0); patch(0); start_wb(0, 0); start_ids(3, 1)

    def steady(i, carry):
        for k in range(_NBUF):
            cc = _PEEL + i * _NBUF + k
            b = k
            bp = 1 - k
            wait_ids(b)
            compute_idx(b)
            wait_wb(b)
            fire_gathers(b)
            wait_gathers(bp)
            patch(bp)
            start_wb(cc - 1, bp)
            start_ids(cc + 2, b)
        return carry

    lax.fori_loop(0, _NSTEADY // _NBUF, steady, 0)

    last = _NCHUNK - 1
    bl = last % _NBUF
    wait_gathers(bl)
    patch(bl)
    start_wb(last, bl)
    wait_ids(_NCHUNK % _NBUF)
    wait_ids((_NCHUNK + 1) % _NBUF)
    for b in range(_NBUF):
        wait_wb(b)

  return _sc_lookup


_sc_a = _make_sc(0)
_sc_b = _make_sc(1)

_BB = 128
_HR = _SEQ * _DIM // 128
_NBLK = _HALF // _BB


def _retile_body(flat_ref, out_ref):
    x = flat_ref[...].reshape(_BB, _HR, 128)
    a = x[:, :, :_DIM]
    b = x[:, :, _DIM:]
    out_ref[pl.Slice(0, _HR, 2), :, :] = jnp.transpose(a, (1, 2, 0))
    out_ref[pl.Slice(1, _HR, 2), :, :] = jnp.transpose(b, (1, 2, 0))


_retile_a = functools.partial(
    pl.pallas_call,
    grid=(_NBLK,),
    in_specs=[pl.BlockSpec((_BB * _HR, 128), lambda i: (i, 0))],
    out_specs=pl.BlockSpec((_SEQ, _DIM, _BB), lambda i: (0, 0, i)),
    out_shape=jax.ShapeDtypeStruct((_SEQ, _DIM, _BATCH), jnp.float32),
)(_retile_body)


def _retile_b_body(flat_ref, prev_ref, out_ref):
    del prev_ref
    _retile_body(flat_ref, out_ref)


_retile_b = functools.partial(
    pl.pallas_call,
    grid=(_NBLK,),
    in_specs=[
        pl.BlockSpec((_BB * _HR, 128), lambda i: (i, 0)),
        pl.BlockSpec(memory_space=pl.ANY),
    ],
    out_specs=pl.BlockSpec((_SEQ, _DIM, _BB), lambda i: (0, 0, i + _NBLK)),
    out_shape=jax.ShapeDtypeStruct((_SEQ, _DIM, _BATCH), jnp.float32),
    input_output_aliases={1: 0},
)(_retile_b_body)


def kernel(input_ids, additional_token_ids, base_table, add_table):
    iid = input_ids.astype(jnp.int32)
    aid = additional_token_ids.astype(jnp.int32)
    lin_a = _sc_a(iid, aid, base_table, add_table)
    lin_b = _sc_b(iid, aid, base_table, add_table)
    o1 = _retile_a(lin_a.reshape(_HALF * _SEQ * _DIM // 128, 128))
    o2 = _retile_b(lin_b.reshape(_HALF * _SEQ * _DIM // 128, 128), o1)
    return o2.transpose(2, 0, 1)
